# baseline (speedup 1.0000x reference)
import jax
import jax.numpy as jnp
from jax.experimental import pallas as pl
from jax.experimental.pallas import tpu as pltpu

_BLK = 512


def _fused_kernel(hid_hbm, out_hbm, attn_ref, sem):
    i = pl.program_id(0)
    copy = pltpu.make_async_copy(hid_hbm, out_hbm, sem)

    @pl.when(i == 0)
    def _start():
        copy.start()

    attn_ref[...] = jnp.zeros_like(attn_ref)

    @pl.when(i == pl.num_programs(0) - 1)
    def _finish():
        copy.wait()


def kernel(hidden_states):
    batch, seq, hid = hidden_states.shape
    h2 = hidden_states.reshape(seq, hid)
    out, attn = pl.pallas_call(
        _fused_kernel,
        grid=(seq // _BLK,),
        in_specs=[pl.BlockSpec(memory_space=pl.ANY)],
        out_specs=[
            pl.BlockSpec(memory_space=pl.ANY),
            pl.BlockSpec((_BLK, seq), lambda i: (i, 0)),
        ],
        out_shape=[
            jax.ShapeDtypeStruct((seq, hid), hidden_states.dtype),
            jax.ShapeDtypeStruct((seq, seq), hidden_states.dtype),
        ],
        scratch_shapes=[pltpu.SemaphoreType.DMA],
    )(h2)
    return (out.reshape(batch, seq, hid), attn.reshape(batch, seq, seq))


# speedup vs baseline: 16.1974x; 16.1974x over previous
import jax
import jax.numpy as jnp
from jax.experimental import pallas as pl

_BLK = 1024


def _copy_zero_kernel(hid_ref, out_ref, attn_ref):
    out_ref[...] = hid_ref[...]
    attn_ref[...] = jnp.zeros_like(attn_ref)


def kernel(hidden_states):
    batch, seq, hid = hidden_states.shape
    h2 = hidden_states.reshape(seq, hid)
    out, attn = pl.pallas_call(
        _copy_zero_kernel,
        grid=(seq // _BLK,),
        in_specs=[pl.BlockSpec((_BLK, hid), lambda i: (i, 0))],
        out_specs=[
            pl.BlockSpec((_BLK, hid), lambda i: (i, 0)),
            pl.BlockSpec((_BLK, seq), lambda i: (i, 0)),
        ],
        out_shape=[
            jax.ShapeDtypeStruct((seq, hid), hidden_states.dtype),
            jax.ShapeDtypeStruct((seq, seq), hidden_states.dtype),
        ],
    )(h2)
    return (out.reshape(batch, seq, hid), attn.reshape(batch, seq, seq))
